# Initial kernel scaffold; baseline (speedup 1.0000x reference)
import jax
import jax.numpy as jnp
from jax.experimental import pallas as pl
from jax.experimental.pallas import tpu as pltpu

RING_LEN = 4096
SLOT = 8
IN_DIM = 128
NUM_CLASSES = 10
GAUSS_K = 2
GAUSS_TAU = 0.5
WALK_PROB = 0.2
B, T = 256, 64
WIN = 2 * GAUSS_K + 1


def _fwd_kernel(xs_ref, Wp_ref, bp_ref, WihT_ref, WhhT_ref, bih_ref,
                bhh_ref, Wj_ref, bj_ref, Wc_ref, bc_ref, w_ref,
                out_ref, hhist_ref, ihist_ref):
    L = RING_LEN
    w = w_ref[0, :]

    def readout(idx, t):
        def body(s, acc):
            d = (idx - ihist_ref[s, :] + GAUSS_K) % L
            wt = jnp.where(d <= (WIN - 1), w[jnp.clip(d, 0, WIN - 1)], 0.0)
            return acc + wt[:, None] * hhist_ref[s, :, :]
        return jax.lax.fori_loop(0, t, body,
                                 jnp.zeros((B, SLOT), dtype=jnp.float32))

    def step(t, ptr):
        xt = xs_ref[t, :, :]
        inp = jnp.dot(xt, Wp_ref[:, :]) + bp_ref[0, :]
        idx = jnp.round(ptr).astype(jnp.int32) % L
        read = readout(idx, t)
        gi = jnp.dot(inp, WihT_ref[:, :]) + bih_ref[0, :]
        gh = jnp.dot(read, WhhT_ref[:, :]) + bhh_ref[0, :]
        i_r = gi[:, 0:SLOT]
        i_z = gi[:, SLOT:2 * SLOT]
        i_n = gi[:, 2 * SLOT:3 * SLOT]
        h_r = gh[:, 0:SLOT]
        h_z = gh[:, SLOT:2 * SLOT]
        h_n = gh[:, 2 * SLOT:3 * SLOT]
        r = jax.nn.sigmoid(i_r + h_r)
        z = jax.nn.sigmoid(i_z + h_z)
        n = jnp.tanh(i_n + r * h_n)
        h = (1.0 - z) * n + z * read
        hhist_ref[t, :, :] = h
        ihist_ref[t, :] = idx
        target = jax.nn.sigmoid(jnp.dot(h, Wj_ref[:, :]) + bj_ref[0, :])[:, 0] * L
        return ((1.0 - WALK_PROB) * target + WALK_PROB * (ptr + 1.0)) % L

    ptr = jax.lax.fori_loop(0, T, step, jnp.zeros((B,), dtype=jnp.float32))
    idx = jnp.round(ptr).astype(jnp.int32) % L
    final = readout(idx, T)
    out_ref[:, :] = jnp.dot(final, Wc_ref[:, :]) + bc_ref[0, :]


def kernel(x, Wp, bp, W_ih, W_hh, b_ih, b_hh, Wj, bj, Wc, bc):
    offs = jnp.arange(-GAUSS_K, GAUSS_K + 1)
    w = jnp.exp(-(offs.astype(jnp.float32) ** 2) / (2.0 * GAUSS_TAU ** 2))
    w = w / w.sum()

    xs = jnp.swapaxes(x, 0, 1)
    out = pl.pallas_call(
        _fwd_kernel,
        out_shape=jax.ShapeDtypeStruct((B, NUM_CLASSES), jnp.float32),
        in_specs=[pl.BlockSpec(memory_space=pltpu.VMEM) for _ in range(12)],
        out_specs=pl.BlockSpec(memory_space=pltpu.VMEM),
        scratch_shapes=[
            pltpu.VMEM((T, B, SLOT), jnp.float32),
            pltpu.VMEM((T, B), jnp.int32),
        ],
    )(xs, Wp, bp.reshape(1, SLOT), W_ih.T, W_hh.T,
      b_ih.reshape(1, 3 * SLOT), b_hh.reshape(1, 3 * SLOT), Wj,
      bj.reshape(1, 1), Wc, bc.reshape(1, NUM_CLASSES), w.reshape(1, WIN))
    return out


# speedup vs baseline: 6.4834x; 6.4834x over previous
import jax
import jax.numpy as jnp
from jax.experimental import pallas as pl
from jax.experimental.pallas import tpu as pltpu

RING_LEN = 4096
SLOT = 8
IN_DIM = 128
NUM_CLASSES = 10
GAUSS_K = 2
GAUSS_TAU = 0.5
WALK_PROB = 0.2
B, T = 256, 64
WIN = 2 * GAUSS_K + 1


def _fwd_kernel(xs_ref, Wp_ref, bp_ref, WihT_ref, WhhT_ref, bih_ref,
                bhh_ref, Wj_ref, bj_ref, Wc_ref, bc_ref, w_ref,
                out_ref, hhist_ref, ihist_ref):
    L = RING_LEN

    def readout(idx, t):
        def body(s, acc):
            d = (idx - ihist_ref[s]) + GAUSS_K
            d = jax.lax.rem(d, L) + jnp.where(d < 0, L, 0)
            wt = jnp.zeros((B, 1), dtype=jnp.float32)
            for i in range(WIN):
                wt = jnp.where(d == i, w_ref[i], wt)
            return acc + wt * hhist_ref[s]
        return jax.lax.fori_loop(0, t, body,
                                 jnp.zeros((B, SLOT), dtype=jnp.float32))

    def step(t, ptr):
        xt = xs_ref[t]
        inp = jnp.dot(xt, Wp_ref[:, :]) + bp_ref[:, :]
        idx = jnp.round(ptr).astype(jnp.int32) % L
        read = readout(idx, t)
        gi = jnp.dot(inp, WihT_ref[:, :]) + bih_ref[:, :]
        gh = jnp.dot(read, WhhT_ref[:, :]) + bhh_ref[:, :]
        i_r = gi[:, 0:SLOT]
        i_z = gi[:, SLOT:2 * SLOT]
        i_n = gi[:, 2 * SLOT:3 * SLOT]
        h_r = gh[:, 0:SLOT]
        h_z = gh[:, SLOT:2 * SLOT]
        h_n = gh[:, 2 * SLOT:3 * SLOT]
        r = jax.nn.sigmoid(i_r + h_r)
        z = jax.nn.sigmoid(i_z + h_z)
        n = jnp.tanh(i_n + r * h_n)
        h = (1.0 - z) * n + z * read
        hhist_ref[t] = h
        ihist_ref[t] = idx
        target = jax.nn.sigmoid(jnp.dot(h, Wj_ref[:, :]) + bj_ref[:, :]) * L
        return ((1.0 - WALK_PROB) * target + WALK_PROB * (ptr + 1.0)) % L

    ptr = jax.lax.fori_loop(0, T, step,
                            jnp.zeros((B, 1), dtype=jnp.float32))
    idx = jnp.round(ptr).astype(jnp.int32) % L
    final = readout(idx, T)
    out_ref[:, :] = jnp.dot(final, Wc_ref[:, :]) + bc_ref[:, :]


def kernel(x, Wp, bp, W_ih, W_hh, b_ih, b_hh, Wj, bj, Wc, bc):
    offs = jnp.arange(-GAUSS_K, GAUSS_K + 1)
    w = jnp.exp(-(offs.astype(jnp.float32) ** 2) / (2.0 * GAUSS_TAU ** 2))
    w = w / w.sum()

    xs = jnp.swapaxes(x, 0, 1)
    vmem = pl.BlockSpec(memory_space=pltpu.VMEM)
    smem = pl.BlockSpec(memory_space=pltpu.SMEM)
    out = pl.pallas_call(
        _fwd_kernel,
        out_shape=jax.ShapeDtypeStruct((B, NUM_CLASSES), jnp.float32),
        in_specs=[vmem] * 11 + [smem],
        out_specs=vmem,
        scratch_shapes=[
            pltpu.VMEM((T, B, SLOT), jnp.float32),
            pltpu.VMEM((T, B, 1), jnp.int32),
        ],
    )(xs, Wp, bp.reshape(1, SLOT), W_ih.T, W_hh.T,
      b_ih.reshape(1, 3 * SLOT), b_hh.reshape(1, 3 * SLOT), Wj,
      bj.reshape(1, 1), Wc, bc.reshape(1, NUM_CLASSES), w)
    return out


# speedup vs baseline: 33.4340x; 5.1569x over previous
import jax
import jax.numpy as jnp
from jax.experimental import pallas as pl
from jax.experimental.pallas import tpu as pltpu

RING_LEN = 4096
SLOT = 8
IN_DIM = 128
NUM_CLASSES = 10
GAUSS_K = 2
GAUSS_TAU = 0.5
WALK_PROB = 0.2
B, T = 256, 64
WIN = 2 * GAUSS_K + 1


def _fwd_kernel(xsT_ref, WpT_ref, bpT_ref, Wih_ref, Whh_ref, bihT_ref,
                bhhT_ref, WjT_ref, bj_ref, WcT_ref, bcT_ref, w_ref,
                out_ref, hhist_ref, ihist_ref):
    L = RING_LEN

    def readout(idx, t):
        def body(s, acc):
            d = jnp.bitwise_and(idx - ihist_ref[s] + GAUSS_K, L - 1)
            wt = jnp.zeros((1, B), dtype=jnp.float32)
            for i in range(WIN):
                wt = jnp.where(d == i, w_ref[i], wt)
            return acc + wt * hhist_ref[s]
        return jax.lax.fori_loop(0, t, body,
                                 jnp.zeros((SLOT, B), dtype=jnp.float32))

    def step(t, ptr):
        xt = xsT_ref[t]
        inp = jnp.dot(WpT_ref[:, :], xt) + bpT_ref[:, :]
        idx = jnp.round(ptr).astype(jnp.int32) % L
        read = readout(idx, t)
        gi = jnp.dot(Wih_ref[:, :], inp) + bihT_ref[:, :]
        gh = jnp.dot(Whh_ref[:, :], read) + bhhT_ref[:, :]
        r = jax.nn.sigmoid(gi[0:SLOT] + gh[0:SLOT])
        z = jax.nn.sigmoid(gi[SLOT:2 * SLOT] + gh[SLOT:2 * SLOT])
        n = jnp.tanh(gi[2 * SLOT:3 * SLOT] + r * gh[2 * SLOT:3 * SLOT])
        h = (1.0 - z) * n + z * read
        hhist_ref[t] = h
        ihist_ref[t] = idx
        target = jax.nn.sigmoid(jnp.dot(WjT_ref[:, :], h) + bj_ref[:, :]) * L
        return ((1.0 - WALK_PROB) * target + WALK_PROB * (ptr + 1.0)) % L

    ptr = jax.lax.fori_loop(0, T, step,
                            jnp.zeros((1, B), dtype=jnp.float32))
    idx = jnp.round(ptr).astype(jnp.int32) % L
    final = readout(idx, T)
    out_ref[:, :] = jnp.dot(WcT_ref[:, :], final) + bcT_ref[:, :]


def kernel(x, Wp, bp, W_ih, W_hh, b_ih, b_hh, Wj, bj, Wc, bc):
    offs = jnp.arange(-GAUSS_K, GAUSS_K + 1)
    w = jnp.exp(-(offs.astype(jnp.float32) ** 2) / (2.0 * GAUSS_TAU ** 2))
    w = w / w.sum()

    xsT = jnp.transpose(x, (1, 2, 0))
    vmem = pl.BlockSpec(memory_space=pltpu.VMEM)
    smem = pl.BlockSpec(memory_space=pltpu.SMEM)
    outT = pl.pallas_call(
        _fwd_kernel,
        out_shape=jax.ShapeDtypeStruct((NUM_CLASSES, B), jnp.float32),
        in_specs=[vmem] * 11 + [smem],
        out_specs=vmem,
        scratch_shapes=[
            pltpu.VMEM((T, SLOT, B), jnp.float32),
            pltpu.VMEM((T, 1, B), jnp.int32),
        ],
    )(xsT, Wp.T, bp.reshape(SLOT, 1), W_ih, W_hh,
      b_ih.reshape(3 * SLOT, 1), b_hh.reshape(3 * SLOT, 1), Wj.T,
      bj.reshape(1, 1), Wc.T, bc.reshape(NUM_CLASSES, 1), w)
    return outT.T


# speedup vs baseline: 35.3136x; 1.0562x over previous
import jax
import jax.numpy as jnp
from jax.experimental import pallas as pl
from jax.experimental.pallas import tpu as pltpu

RING_LEN = 4096
SLOT = 8
IN_DIM = 128
NUM_CLASSES = 10
GAUSS_K = 2
GAUSS_TAU = 0.5
WALK_PROB = 0.2
B, T = 256, 64
WIN = 2 * GAUSS_K + 1
UNROLL = 4


def _fwd_kernel(xsT_ref, WpT_ref, bpT_ref, Wih_ref, Whh_ref, bihT_ref,
                bhhT_ref, WjT_ref, bj_ref, WcT_ref, bcT_ref, w_ref,
                out_ref, hhist_ref, ihist_ref, gihist_ref):
    L = RING_LEN
    HALF = L // 2

    def project(t, _):
        inp = jnp.dot(WpT_ref[:, :], xsT_ref[t]) + bpT_ref[:, :]
        gihist_ref[t] = jnp.dot(Wih_ref[:, :], inp) + bihT_ref[:, :]
        return 0

    jax.lax.fori_loop(0, T, project, 0)

    def readout(idx, t):
        def contrib(acc, s):
            e = jnp.abs(jnp.bitwise_and(idx - ihist_ref[s] + HALF, L - 1)
                        - HALF)
            wt = jnp.where(e == 0, w_ref[GAUSS_K],
                           jnp.where(e == 1, w_ref[GAUSS_K + 1],
                                     jnp.where(e == 2, w_ref[GAUSS_K + 2],
                                               0.0)))
            return acc + wt * hhist_ref[s]

        def body4(j, acc):
            s = j * UNROLL
            for k in range(UNROLL):
                acc = contrib(acc, s + k)
            return acc

        def body1(s, acc):
            return contrib(acc, s)

        acc = jax.lax.fori_loop(0, t // UNROLL, body4,
                                jnp.zeros((SLOT, B), dtype=jnp.float32))
        return jax.lax.fori_loop((t // UNROLL) * UNROLL, t, body1, acc)

    def step(t, ptr):
        idx = jnp.round(ptr).astype(jnp.int32) % L
        read = readout(idx, t)
        gi = gihist_ref[t]
        gh = jnp.dot(Whh_ref[:, :], read) + bhhT_ref[:, :]
        r = jax.nn.sigmoid(gi[0:SLOT] + gh[0:SLOT])
        z = jax.nn.sigmoid(gi[SLOT:2 * SLOT] + gh[SLOT:2 * SLOT])
        n = jnp.tanh(gi[2 * SLOT:3 * SLOT] + r * gh[2 * SLOT:3 * SLOT])
        h = (1.0 - z) * n + z * read
        hhist_ref[t] = h
        ihist_ref[t] = idx
        target = jax.nn.sigmoid(jnp.dot(WjT_ref[:, :], h) + bj_ref[:, :]) * L
        return ((1.0 - WALK_PROB) * target + WALK_PROB * (ptr + 1.0)) % L

    ptr = jax.lax.fori_loop(0, T, step,
                            jnp.zeros((1, B), dtype=jnp.float32))
    idx = jnp.round(ptr).astype(jnp.int32) % L
    final = readout(idx, T)
    out_ref[:, :] = jnp.dot(WcT_ref[:, :], final) + bcT_ref[:, :]


def kernel(x, Wp, bp, W_ih, W_hh, b_ih, b_hh, Wj, bj, Wc, bc):
    offs = jnp.arange(-GAUSS_K, GAUSS_K + 1)
    w = jnp.exp(-(offs.astype(jnp.float32) ** 2) / (2.0 * GAUSS_TAU ** 2))
    w = w / w.sum()

    xsT = jnp.transpose(x, (1, 2, 0))
    vmem = pl.BlockSpec(memory_space=pltpu.VMEM)
    smem = pl.BlockSpec(memory_space=pltpu.SMEM)
    outT = pl.pallas_call(
        _fwd_kernel,
        out_shape=jax.ShapeDtypeStruct((NUM_CLASSES, B), jnp.float32),
        in_specs=[vmem] * 11 + [smem],
        out_specs=vmem,
        scratch_shapes=[
            pltpu.VMEM((T, SLOT, B), jnp.float32),
            pltpu.VMEM((T, 1, B), jnp.int32),
            pltpu.VMEM((T, 3 * SLOT, B), jnp.float32),
        ],
    )(xsT, Wp.T, bp.reshape(SLOT, 1), W_ih, W_hh,
      b_ih.reshape(3 * SLOT, 1), b_hh.reshape(3 * SLOT, 1), Wj.T,
      bj.reshape(1, 1), Wc.T, bc.reshape(NUM_CLASSES, 1), w)
    return outT.T


# speedup vs baseline: 41.3275x; 1.1703x over previous
import jax
import jax.numpy as jnp
from jax.experimental import pallas as pl
from jax.experimental.pallas import tpu as pltpu

RING_LEN = 4096
SLOT = 8
IN_DIM = 128
NUM_CLASSES = 10
GAUSS_K = 2
GAUSS_TAU = 0.5
WALK_PROB = 0.2
B, T = 256, 64
WIN = 2 * GAUSS_K + 1
UNROLL = 8


def _fwd_kernel(xsT_ref, WpT_ref, bpT_ref, Wih_ref, Whh_ref, bihT_ref,
                bhhT_ref, WjT_ref, bj_ref, WcT_ref, bcT_ref, w_ref,
                out_ref, hhist_ref, ihist_ref, gihist_ref):
    L = RING_LEN
    HALF = L // 2

    def project(t, _):
        inp = jnp.dot(WpT_ref[:, :], xsT_ref[t]) + bpT_ref[:, :]
        gihist_ref[t] = jnp.dot(Wih_ref[:, :], inp) + bihT_ref[:, :]
        return 0

    jax.lax.fori_loop(0, T, project, 0, unroll=4)

    def readout(idx, t):
        idxH = idx + HALF

        def weights(e):
            return jnp.where(e == 0, w_ref[GAUSS_K],
                             jnp.where(e == 1, w_ref[GAUSS_K + 1],
                                       jnp.where(e == 2, w_ref[GAUSS_K + 2],
                                                 0.0)))

        def bodyN(j, acc):
            s = j * UNROLL
            ih = ihist_ref[pl.ds(s, UNROLL), 0, :]
            e = jnp.abs(jnp.bitwise_and(idxH - ih, L - 1) - HALF)
            wt = weights(e)
            hU = hhist_ref[pl.ds(s, UNROLL)]
            for k in range(UNROLL):
                acc = acc + wt[k:k + 1, :] * hU[k]
            return acc

        def body1(s, acc):
            e = jnp.abs(jnp.bitwise_and(idxH - ihist_ref[s], L - 1) - HALF)
            return acc + weights(e) * hhist_ref[s]

        acc = jax.lax.fori_loop(0, t // UNROLL, bodyN,
                                jnp.zeros((SLOT, B), dtype=jnp.float32))
        return jax.lax.fori_loop((t // UNROLL) * UNROLL, t, body1, acc)

    def step(t, ptr):
        idx = jnp.round(ptr).astype(jnp.int32) % L
        read = readout(idx, t)
        gi = gihist_ref[t]
        gh = jnp.dot(Whh_ref[:, :], read) + bhhT_ref[:, :]
        r = jax.nn.sigmoid(gi[0:SLOT] + gh[0:SLOT])
        z = jax.nn.sigmoid(gi[SLOT:2 * SLOT] + gh[SLOT:2 * SLOT])
        n = jnp.tanh(gi[2 * SLOT:3 * SLOT] + r * gh[2 * SLOT:3 * SLOT])
        h = (1.0 - z) * n + z * read
        hhist_ref[t] = h
        ihist_ref[t] = idx
        target = jax.nn.sigmoid(jnp.dot(WjT_ref[:, :], h) + bj_ref[:, :]) * L
        return ((1.0 - WALK_PROB) * target + WALK_PROB * (ptr + 1.0)) % L

    ptr = jax.lax.fori_loop(0, T, step,
                            jnp.zeros((1, B), dtype=jnp.float32))
    idx = jnp.round(ptr).astype(jnp.int32) % L
    final = readout(idx, T)
    out_ref[:, :] = jnp.dot(WcT_ref[:, :], final) + bcT_ref[:, :]


def kernel(x, Wp, bp, W_ih, W_hh, b_ih, b_hh, Wj, bj, Wc, bc):
    offs = jnp.arange(-GAUSS_K, GAUSS_K + 1)
    w = jnp.exp(-(offs.astype(jnp.float32) ** 2) / (2.0 * GAUSS_TAU ** 2))
    w = w / w.sum()

    xsT = jnp.transpose(x, (1, 2, 0))
    vmem = pl.BlockSpec(memory_space=pltpu.VMEM)
    smem = pl.BlockSpec(memory_space=pltpu.SMEM)
    outT = pl.pallas_call(
        _fwd_kernel,
        out_shape=jax.ShapeDtypeStruct((NUM_CLASSES, B), jnp.float32),
        in_specs=[vmem] * 11 + [smem],
        out_specs=vmem,
        scratch_shapes=[
            pltpu.VMEM((T, SLOT, B), jnp.float32),
            pltpu.VMEM((T, 1, B), jnp.int32),
            pltpu.VMEM((T, 3 * SLOT, B), jnp.float32),
        ],
    )(xsT, Wp.T, bp.reshape(SLOT, 1), W_ih, W_hh,
      b_ih.reshape(3 * SLOT, 1), b_hh.reshape(3 * SLOT, 1), Wj.T,
      bj.reshape(1, 1), Wc.T, bc.reshape(NUM_CLASSES, 1), w)
    return outT.T


# speedup vs baseline: 45.1960x; 1.0936x over previous
import jax
import jax.numpy as jnp
from jax.experimental import pallas as pl
from jax.experimental.pallas import tpu as pltpu

RING_LEN = 4096
SLOT = 8
IN_DIM = 128
NUM_CLASSES = 10
GAUSS_K = 2
GAUSS_TAU = 0.5
WALK_PROB = 0.2
B, T = 256, 64
H = B // 2
WIN = 2 * GAUSS_K + 1
UNROLL = 8


def _fwd_kernel(xsT_ref, WpT_ref, bpT_ref, Wih_ref, Whh_ref, bihT_ref,
                bhhT_ref, WjT_ref, bj_ref, WcT_ref, bcT_ref, w_ref,
                out_ref, hhA_ref, hhB_ref, ihA_ref, ihB_ref,
                giA_ref, giB_ref):
    L = RING_LEN
    HALF = L // 2

    def project(t, _):
        inp = jnp.dot(WpT_ref[:, :], xsT_ref[t]) + bpT_ref[:, :]
        gi = jnp.dot(Wih_ref[:, :], inp) + bihT_ref[:, :]
        giA_ref[t] = gi[:, 0:H]
        giB_ref[t] = gi[:, H:B]
        return 0

    jax.lax.fori_loop(0, T, project, 0, unroll=4)

    def readout(idx, t, ih_ref, hh_ref):
        idxH = idx + HALF

        def weights(e):
            return jnp.where(e == 0, w_ref[GAUSS_K],
                             jnp.where(e == 1, w_ref[GAUSS_K + 1],
                                       jnp.where(e == 2, w_ref[GAUSS_K + 2],
                                                 0.0)))

        def bodyN(j, acc):
            s = j * UNROLL
            ih = ih_ref[pl.ds(s, UNROLL), 0, :]
            e = jnp.abs(jnp.bitwise_and(idxH - ih, L - 1) - HALF)
            wt = weights(e)
            hU = hh_ref[pl.ds(s, UNROLL)]
            for k in range(UNROLL):
                acc = acc + wt[k:k + 1, :] * hU[k]
            return acc

        def body1(s, acc):
            e = jnp.abs(jnp.bitwise_and(idxH - ih_ref[s], L - 1) - HALF)
            return acc + weights(e) * hh_ref[s]

        acc = jax.lax.fori_loop(0, t // UNROLL, bodyN,
                                jnp.zeros((SLOT, H), dtype=jnp.float32))
        return jax.lax.fori_loop((t // UNROLL) * UNROLL, t, body1, acc)

    def half_step(t, ptr, ih_ref, hh_ref, gi_ref):
        idx = jnp.round(ptr).astype(jnp.int32) % L
        read = readout(idx, t, ih_ref, hh_ref)
        gi = gi_ref[t]
        gh = jnp.dot(Whh_ref[:, :], read) + bhhT_ref[:, :]
        r = jax.nn.sigmoid(gi[0:SLOT] + gh[0:SLOT])
        z = jax.nn.sigmoid(gi[SLOT:2 * SLOT] + gh[SLOT:2 * SLOT])
        n = jnp.tanh(gi[2 * SLOT:3 * SLOT] + r * gh[2 * SLOT:3 * SLOT])
        h = (1.0 - z) * n + z * read
        hh_ref[t] = h
        ih_ref[t] = idx
        target = jax.nn.sigmoid(jnp.dot(WjT_ref[:, :], h) + bj_ref[:, :]) * L
        return ((1.0 - WALK_PROB) * target + WALK_PROB * (ptr + 1.0)) % L

    def step(t, carry):
        ptrA, ptrB = carry
        ptrA = half_step(t, ptrA, ihA_ref, hhA_ref, giA_ref)
        ptrB = half_step(t, ptrB, ihB_ref, hhB_ref, giB_ref)
        return (ptrA, ptrB)

    zero = jnp.zeros((1, H), dtype=jnp.float32)
    ptrA, ptrB = jax.lax.fori_loop(0, T, step, (zero, zero))
    idxA = jnp.round(ptrA).astype(jnp.int32) % L
    idxB = jnp.round(ptrB).astype(jnp.int32) % L
    finalA = readout(idxA, T, ihA_ref, hhA_ref)
    finalB = readout(idxB, T, ihB_ref, hhB_ref)
    out_ref[:, 0:H] = jnp.dot(WcT_ref[:, :], finalA) + bcT_ref[:, :]
    out_ref[:, H:B] = jnp.dot(WcT_ref[:, :], finalB) + bcT_ref[:, :]


def kernel(x, Wp, bp, W_ih, W_hh, b_ih, b_hh, Wj, bj, Wc, bc):
    offs = jnp.arange(-GAUSS_K, GAUSS_K + 1)
    w = jnp.exp(-(offs.astype(jnp.float32) ** 2) / (2.0 * GAUSS_TAU ** 2))
    w = w / w.sum()

    xsT = jnp.transpose(x, (1, 2, 0))
    vmem = pl.BlockSpec(memory_space=pltpu.VMEM)
    smem = pl.BlockSpec(memory_space=pltpu.SMEM)
    outT = pl.pallas_call(
        _fwd_kernel,
        out_shape=jax.ShapeDtypeStruct((NUM_CLASSES, B), jnp.float32),
        in_specs=[vmem] * 11 + [smem],
        out_specs=vmem,
        scratch_shapes=[
            pltpu.VMEM((T, SLOT, H), jnp.float32),
            pltpu.VMEM((T, SLOT, H), jnp.float32),
            pltpu.VMEM((T, 1, H), jnp.int32),
            pltpu.VMEM((T, 1, H), jnp.int32),
            pltpu.VMEM((T, 3 * SLOT, H), jnp.float32),
            pltpu.VMEM((T, 3 * SLOT, H), jnp.float32),
        ],
    )(xsT, Wp.T, bp.reshape(SLOT, 1), W_ih, W_hh,
      b_ih.reshape(3 * SLOT, 1), b_hh.reshape(3 * SLOT, 1), Wj.T,
      bj.reshape(1, 1), Wc.T, bc.reshape(NUM_CLASSES, 1), w)
    return outT.T
